# baseline (speedup 1.0000x reference)
import functools

import jax
import jax.numpy as jnp
from jax import lax
from jax.experimental import pallas as pl
from jax.experimental.pallas import tpu as pltpu
from jax.experimental.pallas import tpu_sc as plsc

EMBED = 384
NH = 8
NP = 12
NL = 8
HS = 32
WS = 32
HW = HS * WS
DPH = EMBED // NH
HD = DPH // NH
G = 12
IMGS = G * NH
F32 = jnp.float32


def _proj_body(x_ref, xv_ref, pos_ref, w1_ref, b1_ref, w2_ref, b2_ref,
               pix_ref, ex0_ref, ex1_ref, fy1_ref, v_ref):
    q = x_ref[0] + pos_ref[0]
    o1 = jnp.dot(w1_ref[0], q, preferred_element_type=F32) + b1_ref[0]
    sox = o1[:NH * NP]
    soy = o1[NH * NP:2 * NH * NP]
    a = o1[2 * NH * NP:]
    a = a - jnp.max(a, axis=0, keepdims=True)
    ex = jnp.exp(a)
    hsel = (lax.broadcasted_iota(jnp.int32, (NH, NH * NP), 1) // NP
            == lax.broadcasted_iota(jnp.int32, (NH, NH * NP), 0)).astype(F32)
    rinv = 1.0 / jnp.dot(hsel, ex, preferred_element_type=F32)
    ep = ex * jnp.dot(hsel.T, rinv, preferred_element_type=F32)
    qpos = lax.broadcasted_iota(jnp.int32, (1, HW), 1)
    colf = (qpos % WS).astype(F32)
    rowf = (qpos // WS).astype(F32)
    xs = jnp.clip(colf + sox + 1.0, 0.0, 33.0)
    ys = jnp.clip(rowf + soy + 1.0, 0.0, 33.0)
    x0 = jnp.floor(xs)
    y0 = jnp.floor(ys)
    pix_ref[0] = (y0 * PW + x0).astype(jnp.int32)
    ex1 = (xs - x0) * ep
    ex0_ref[0] = ep - ex1
    ex1_ref[0] = ex1
    fy1_ref[0] = ys - y0
    v_ref[0] = jnp.dot(w2_ref[0], xv_ref[0], preferred_element_type=F32) + b2_ref[0]


def _out_body(s_ref, x_ref, w_ref, b_ref, o_ref):
    o_ref[0] = (jnp.dot(w_ref[0], s_ref[0], preferred_element_type=F32)
                + b_ref[0] + x_ref[0])


def _proj_call(x, posT, w1, b1, w2, b2):
    n1 = NH * NP * 2 + NH * NP
    return pl.pallas_call(
        _proj_body,
        grid=(G,),
        in_specs=[
            pl.BlockSpec((1, EMBED, HW), lambda g: (g, 0, 0)),
            pl.BlockSpec((1, EMBED, HW), lambda g: ((g + 6) % 12, 0, 0)),
            pl.BlockSpec((1, EMBED, HW), lambda g: (g // 6, 0, 0)),
            pl.BlockSpec((1, n1, EMBED), lambda g: (g // 6, 0, 0)),
            pl.BlockSpec((1, n1, 1), lambda g: (g // 6, 0, 0)),
            pl.BlockSpec((1, DPH, EMBED), lambda g: (g // 6, 0, 0)),
            pl.BlockSpec((1, DPH, 1), lambda g: (g // 6, 0, 0)),
        ],
        out_specs=[
            pl.BlockSpec((1, NH * NP, HW), lambda g: (g, 0, 0)),
            pl.BlockSpec((1, NH * NP, HW), lambda g: (g, 0, 0)),
            pl.BlockSpec((1, NH * NP, HW), lambda g: (g, 0, 0)),
            pl.BlockSpec((1, NH * NP, HW), lambda g: (g, 0, 0)),
            pl.BlockSpec((1, DPH, HW), lambda g: (g, 0, 0)),
        ],
        out_shape=[
            jax.ShapeDtypeStruct((G, NH * NP, HW), jnp.int32),
            jax.ShapeDtypeStruct((G, NH * NP, HW), F32),
            jax.ShapeDtypeStruct((G, NH * NP, HW), F32),
            jax.ShapeDtypeStruct((G, NH * NP, HW), F32),
            jax.ShapeDtypeStruct((G, DPH, HW), F32),
        ],
    )(x, x, posT, w1, b1, w2, b2)


def _out_call(samp, x, w, b):
    return pl.pallas_call(
        _out_body,
        grid=(G,),
        in_specs=[
            pl.BlockSpec((1, DPH, HW), lambda g: (g, 0, 0)),
            pl.BlockSpec((1, EMBED, HW), lambda g: (g, 0, 0)),
            pl.BlockSpec((1, EMBED, DPH), lambda g: (g // 6, 0, 0)),
            pl.BlockSpec((1, EMBED, 1), lambda g: (g // 6, 0, 0)),
        ],
        out_specs=pl.BlockSpec((1, EMBED, HW), lambda g: (g, 0, 0)),
        out_shape=jax.ShapeDtypeStruct((G, EMBED, HW), F32),
    )(samp, x, w, b)


PH = NP // 2
PW = 36
PIMG = PW * PW
PTAB = HD * PIMG


@functools.lru_cache(maxsize=None)
def _sc_sampler():
    info = plsc.get_sparse_core_info()
    NC, NS, L = info.num_cores, info.num_subcores, info.num_lanes
    NW = NC * NS
    per_w = IMGS // NW
    nblk = HW // L
    mesh = plsc.VectorSubcoreMesh(core_axis_name="c", subcore_axis_name="s")

    @functools.partial(
        pl.kernel,
        mesh=mesh,
        out_type=jax.ShapeDtypeStruct((IMGS, HD * HW), F32),
        compiler_params=pltpu.CompilerParams(needs_layout_passes=False,
                                             disable_bounds_checks=True),
        scratch_types=[
            pltpu.VMEM((PTAB,), F32),
            pltpu.VMEM((PTAB,), F32),
            pltpu.VMEM((PH * HW,), jnp.int32),
            pltpu.VMEM((PH * HW,), jnp.int32),
            pltpu.VMEM((PH * HW,), F32),
            pltpu.VMEM((PH * HW,), F32),
            pltpu.VMEM((PH * HW,), F32),
            pltpu.VMEM((PH * HW,), F32),
            pltpu.VMEM((PH * HW,), F32),
            pltpu.VMEM((PH * HW,), F32),
            pltpu.VMEM((HD * HW,), F32),
            pltpu.SemaphoreType.DMA,
            pltpu.SemaphoreType.DMA,
            pltpu.SemaphoreType.DMA,
            pltpu.SemaphoreType.DMA,
        ],
    )
    def _sc_sample(v_hbm, px_hbm, e0_hbm, e1_hbm, fy_hbm, out_hbm,
                   v_0, v_1, px_0, px_1, e0_0, e0_1, e1_0, e1_1, fy_0, fy_1,
                   o_v, sem0, sem1, semv0, semv1):
        wid = lax.axis_index("s") * NC + lax.axis_index("c")
        vbufs = ((v_0, semv0), (v_1, semv1))
        cbufs = ((px_0, e0_0, e1_0, fy_0, sem0),
                 (px_1, e0_1, e1_1, fy_1, sem1))
        steps = per_w * 2

        def startv(k):
            vb, sem = vbufs[k % 2]
            return (pltpu.async_copy(v_hbm.at[wid * per_w + k], vb, sem),)

        def startc(s):
            k, j = divmod(s, 2)
            m = wid * per_w + k
            pb, e0b, e1b, fyb, sem = cbufs[s % 2]
            sl = pl.ds(j * PH * HW, PH * HW)
            return (pltpu.async_copy(px_hbm.at[m, sl], pb, sem),
                    pltpu.async_copy(e0_hbm.at[m, sl], e0b, sem),
                    pltpu.async_copy(e1_hbm.at[m, sl], e1b, sem),
                    pltpu.async_copy(fy_hbm.at[m, sl], fyb, sem))

        pendv = {0: startv(0)}
        pendc = {0: startc(0)}
        for s in range(steps):
            k, j = divmod(s, 2)
            if s + 1 < steps:
                pendc[s + 1] = startc(s + 1)
            if j == 0 and k + 1 < per_w:
                pendv[k + 1] = startv(k + 1)
            if j == 0:
                for h in pendv.pop(k):
                    h.wait()
            for h in pendc.pop(s):
                h.wait()
            v_v, _ = vbufs[k % 2]
            px_v, e0_v, e1_v, fy_v, _ = cbufs[s % 2]
            taps = [v_v.at[pl.ds(c * PIMG, PIMG)] for c in range(HD)]

            def blk(i, carry2, j=j, taps=taps,
                    px_v=px_v, e0_v=e0_v, e1_v=e1_v, fy_v=fy_v):
                base = i * L
                acc = [jnp.zeros((L,), F32) for _ in range(HD)]
                for p in range(PH):
                    e0 = e0_v[pl.ds(p * HW + base, L)]
                    e1 = e1_v[pl.ds(p * HW + base, L)]
                    fy = fy_v[pl.ds(p * HW + base, L)]
                    pix = px_v[pl.ds(p * HW + base, L)]
                    w10 = e0 * fy
                    w11 = e1 * fy
                    wts = (e0 - w10, e1 - w11, w10, w11)
                    idxs = (pix, pix + 1, pix + PW, pix + (PW + 1))
                    for c in range(HD):
                        a = acc[c]
                        for t in range(4):
                            a = a + wts[t] * plsc.load_gather(taps[c], [idxs[t]])
                        acc[c] = a
                for c in range(HD):
                    o = pl.ds(c * HW + base, L)
                    if j == 0:
                        o_v[o] = acc[c]
                    else:
                        o_v[o] = o_v[o] + acc[c]
                return carry2

            lax.fori_loop(0, nblk, blk, 0)
            if j == 1:
                pltpu.sync_copy(o_v, out_hbm.at[wid * per_w + k])

    return _sc_sample


def _posT_one(re_s, ce_s):
    pe = jnp.concatenate([
        jnp.broadcast_to(ce_s[None, :, :], (HS, WS, EMBED // 2)),
        jnp.broadcast_to(re_s[:, None, :], (HS, WS, EMBED // 2))], -1)
    return pe.reshape(HW, EMBED).T


def kernel(rgb_fea, ir_fea, so_W, so_b, aw_W, aw_b, vp_W, vp_b, op_W, op_b,
           row_embed, col_embed):
    xr = rgb_fea.transpose(0, 2, 1, 3, 4).reshape(6, EMBED, HW)
    xi = ir_fea.transpose(0, 2, 1, 3, 4).reshape(6, EMBED, HW)
    x = jnp.concatenate([xr, xi], axis=0)

    posT = jnp.stack([_posT_one(row_embed[0], col_embed[0]),
                      _posT_one(row_embed[1], col_embed[1])])

    w1 = jnp.swapaxes(jnp.concatenate(
        [so_W[..., 0::2], so_W[..., 1::2], aw_W], -1), -1, -2)
    b1 = jnp.concatenate(
        [so_b[..., 0::2], so_b[..., 1::2], aw_b], -1)[..., None]
    w2 = jnp.swapaxes(vp_W, -1, -2)
    b2 = vp_b[..., None]
    w3 = jnp.swapaxes(op_W, -1, -2)
    b3 = op_b[..., None]

    sampler = _sc_sampler()
    for l in range(NL):
        pix_t, ex0_t, ex1_t, fy1_t, v_t = _proj_call(x, posT, w1[:, l],
                                                     b1[:, l], w2[:, l],
                                                     b2[:, l])
        v_pad = jnp.pad(v_t.reshape(IMGS, HD, HS, WS),
                        ((0, 0), (0, 0), (1, 3), (1, 3))).reshape(IMGS, PTAB)
        samp = sampler(v_pad,
                       pix_t.reshape(IMGS, NP * HW),
                       ex0_t.reshape(IMGS, NP * HW),
                       ex1_t.reshape(IMGS, NP * HW),
                       fy1_t.reshape(IMGS, NP * HW))
        x = _out_call(samp.reshape(G, DPH, HW), x, w3[:, l], b3[:, l])

    y = x.reshape(2, 2, 3, EMBED, HS, WS).transpose(0, 1, 3, 2, 4, 5)
    return y


# speedup vs baseline: 1.0807x; 1.0807x over previous
import functools

import jax
import jax.numpy as jnp
from jax import lax
from jax.experimental import pallas as pl
from jax.experimental.pallas import tpu as pltpu
from jax.experimental.pallas import tpu_sc as plsc

EMBED = 384
NH = 8
NP = 12
NL = 8
HS = 32
WS = 32
HW = HS * WS
DPH = EMBED // NH
HD = DPH // NH
G = 12
IMGS = G * NH
F32 = jnp.float32

NSO = NH * NP * 2
NAW = NH * NP
NPK = NSO + NAW + DPH

PW = 36
PIMG = PW * PW
PTAB = HD * PIMG


def _proj_body(x_ref, xv_ref, pos_ref, w1_ref, b1_ref, w2_ref, b2_ref,
               pk_ref):
    q = x_ref[0] + pos_ref[0]
    o1 = jnp.dot(w1_ref[0], q, preferred_element_type=F32) + b1_ref[0]
    a = o1[NSO:]
    a = a - jnp.max(a, axis=0, keepdims=True)
    ex = jnp.exp(a)
    hsel = (lax.broadcasted_iota(jnp.int32, (NH, NAW), 1) // NP
            == lax.broadcasted_iota(jnp.int32, (NH, NAW), 0)).astype(F32)
    rinv = 1.0 / jnp.dot(hsel, ex, preferred_element_type=F32)
    ep = ex * jnp.dot(hsel.T, rinv, preferred_element_type=F32)
    v = jnp.dot(w2_ref[0], xv_ref[0], preferred_element_type=F32) + b2_ref[0]
    pk_ref[0] = jnp.concatenate([o1[:NSO], ep, v], axis=0)


def _out_body(s_ref, x_ref, w_ref, b_ref, o_ref):
    o_ref[0] = (jnp.dot(w_ref[0], s_ref[0], preferred_element_type=F32)
                + b_ref[0] + x_ref[0])


def _proj_call(x, posT, w1, b1, w2, b2):
    n1 = NSO + NAW
    return pl.pallas_call(
        _proj_body,
        grid=(G,),
        in_specs=[
            pl.BlockSpec((1, EMBED, HW), lambda g: (g, 0, 0)),
            pl.BlockSpec((1, EMBED, HW), lambda g: ((g + 6) % 12, 0, 0)),
            pl.BlockSpec((1, EMBED, HW), lambda g: (g // 6, 0, 0)),
            pl.BlockSpec((1, n1, EMBED), lambda g: (g // 6, 0, 0)),
            pl.BlockSpec((1, n1, 1), lambda g: (g // 6, 0, 0)),
            pl.BlockSpec((1, DPH, EMBED), lambda g: (g // 6, 0, 0)),
            pl.BlockSpec((1, DPH, 1), lambda g: (g // 6, 0, 0)),
        ],
        out_specs=pl.BlockSpec((1, NPK, HW), lambda g: (g, 0, 0)),
        out_shape=jax.ShapeDtypeStruct((G, NPK, HW), F32),
    )(x, x, posT, w1, b1, w2, b2)


def _out_call(samp, x, w, b):
    return pl.pallas_call(
        _out_body,
        grid=(G,),
        in_specs=[
            pl.BlockSpec((1, DPH, HW), lambda g: (g, 0, 0)),
            pl.BlockSpec((1, EMBED, HW), lambda g: (g, 0, 0)),
            pl.BlockSpec((1, EMBED, DPH), lambda g: (g // 6, 0, 0)),
            pl.BlockSpec((1, EMBED, 1), lambda g: (g // 6, 0, 0)),
        ],
        out_specs=pl.BlockSpec((1, EMBED, HW), lambda g: (g, 0, 0)),
        out_shape=jax.ShapeDtypeStruct((G, EMBED, HW), F32),
    )(samp, x, w, b)


@functools.lru_cache(maxsize=None)
def _sc_sampler():
    info = plsc.get_sparse_core_info()
    NC, NS, L = info.num_cores, info.num_subcores, info.num_lanes
    NW = NC * NS
    per_w = IMGS // NW
    nblk = HW // L
    mesh = plsc.VectorSubcoreMesh(core_axis_name="c", subcore_axis_name="s")

    @functools.partial(
        pl.kernel,
        mesh=mesh,
        out_type=jax.ShapeDtypeStruct((IMGS, HD * HW), F32),
        compiler_params=pltpu.CompilerParams(needs_layout_passes=False,
                                             disable_bounds_checks=True),
        scratch_types=[
            pltpu.VMEM((PTAB,), F32),
            pltpu.VMEM((HD * HW,), F32),
            pltpu.VMEM((HD * HW,), F32),
            pltpu.VMEM((NP * 2 * HW,), F32),
            pltpu.VMEM((NP * 2 * HW,), F32),
            pltpu.VMEM((NP * HW,), F32),
            pltpu.VMEM((NP * HW,), F32),
            pltpu.VMEM((HD * HW,), F32),
            pltpu.SemaphoreType.DMA,
            pltpu.SemaphoreType.DMA,
        ],
    )
    def _sc_sample(pk_hbm, out_hbm,
                   v_pad, vr_0, vr_1, so_0, so_1, aw_0, aw_1, o_v,
                   sem0, sem1):
        wid = lax.axis_index("s") * NC + lax.axis_index("c")
        lane = lax.iota(jnp.int32, L)
        bufs = ((vr_0, so_0, aw_0, sem0), (vr_1, so_1, aw_1, sem1))

        def zblk(i, c):
            v_pad[pl.ds(i * L, L)] = jnp.zeros((L,), F32)
            return c

        lax.fori_loop(0, PTAB // L, zblk, 0)

        def start(k):
            m = wid * per_w + k
            g = m // NH
            h = m % NH
            vb, sb, ab, sem = bufs[k % 2]
            so_off = (h * NP * 2) * HW
            aw_off = (NSO + h * NP) * HW
            v_off = (NSO + NAW + h * HD) * HW
            return (pltpu.async_copy(
                        pk_hbm.at[g, pl.ds(so_off, NP * 2 * HW)], sb, sem),
                    pltpu.async_copy(
                        pk_hbm.at[g, pl.ds(aw_off, NP * HW)], ab, sem),
                    pltpu.async_copy(
                        pk_hbm.at[g, pl.ds(v_off, HD * HW)], vb, sem))

        pend = {0: start(0)}
        for k in range(per_w):
            vr_v, so_v, aw_v, _ = bufs[k % 2]
            if k + 1 < per_w:
                pend[k + 1] = start(k + 1)
            for hc in pend.pop(k):
                hc.wait()

            def prow(r, c0, vr_v=vr_v):
                for c in range(HD):
                    for j in range(2):
                        src = vr_v[pl.ds(c * HW + r * WS + j * L, L)]
                        didx = (r * PW
                                + (c * PIMG + PW + 1 + j * L) + lane)
                        plsc.store_scatter(v_pad, [didx], src)
                return c0

            lax.fori_loop(0, HS, prow, 0)

            taps = [v_pad.at[pl.ds(c * PIMG, PIMG)] for c in range(HD)]

            def blk(i, carry2, so_v=so_v, aw_v=aw_v, taps=taps):
                base = i * L
                rowf = (base // WS).astype(F32)
                colf = ((base % WS) + lane).astype(F32)
                acc = [jnp.zeros((L,), F32) for _ in range(HD)]
                for p in range(NP):
                    ep = aw_v[pl.ds(p * HW + base, L)]
                    xs = jnp.minimum(jnp.maximum(
                        colf + so_v[pl.ds(2 * p * HW + base, L)] + 1.0,
                        0.0), 33.0)
                    ys = jnp.minimum(jnp.maximum(
                        rowf + so_v[pl.ds((2 * p + 1) * HW + base, L)] + 1.0,
                        0.0), 33.0)
                    x0i = xs.astype(jnp.int32)
                    y0i = ys.astype(jnp.int32)
                    fx1 = xs - x0i.astype(F32)
                    fx0 = 1.0 - fx1
                    fy1 = ys - y0i.astype(F32)
                    fy0 = 1.0 - fy1
                    pix = y0i * PW + x0i
                    idxs = (pix, pix + 1, pix + PW, pix + (PW + 1))
                    ex0 = fx0 * ep
                    ex1 = fx1 * ep
                    wts = (ex0 * fy0, ex1 * fy0, ex0 * fy1, ex1 * fy1)
                    for c in range(HD):
                        a = acc[c]
                        for t in range(4):
                            a = a + wts[t] * plsc.load_gather(taps[c],
                                                              [idxs[t]])
                        acc[c] = a
                for c in range(HD):
                    o_v[pl.ds(c * HW + base, L)] = acc[c]
                return carry2

            lax.fori_loop(0, nblk, blk, 0)
            pltpu.sync_copy(o_v, out_hbm.at[wid * per_w + k])

    return _sc_sample


def _posT_one(re_s, ce_s):
    pe = jnp.concatenate([
        jnp.broadcast_to(ce_s[None, :, :], (HS, WS, EMBED // 2)),
        jnp.broadcast_to(re_s[:, None, :], (HS, WS, EMBED // 2))], -1)
    return pe.reshape(HW, EMBED).T


def kernel(rgb_fea, ir_fea, so_W, so_b, aw_W, aw_b, vp_W, vp_b, op_W, op_b,
           row_embed, col_embed):
    xr = rgb_fea.transpose(0, 2, 1, 3, 4).reshape(6, EMBED, HW)
    xi = ir_fea.transpose(0, 2, 1, 3, 4).reshape(6, EMBED, HW)
    x = jnp.concatenate([xr, xi], axis=0)

    posT = jnp.stack([_posT_one(row_embed[0], col_embed[0]),
                      _posT_one(row_embed[1], col_embed[1])])

    w1 = jnp.swapaxes(jnp.concatenate([so_W, aw_W], -1), -1, -2)
    b1 = jnp.concatenate([so_b, aw_b], -1)[..., None]
    w2 = jnp.swapaxes(vp_W, -1, -2)
    b2 = vp_b[..., None]
    w3 = jnp.swapaxes(op_W, -1, -2)
    b3 = op_b[..., None]

    sampler = _sc_sampler()
    for l in range(NL):
        pk = _proj_call(x, posT, w1[:, l], b1[:, l], w2[:, l], b2[:, l])
        samp = sampler(pk.reshape(G, NPK * HW))
        x = _out_call(samp.reshape(G, DPH, HW), x, w3[:, l], b3[:, l])

    y = x.reshape(2, 2, 3, EMBED, HS, WS).transpose(0, 1, 3, 2, 4, 5)
    return y


# speedup vs baseline: 1.4388x; 1.3313x over previous
import functools

import jax
import jax.numpy as jnp
from jax import lax
from jax.experimental import pallas as pl
from jax.experimental.pallas import tpu as pltpu
from jax.experimental.pallas import tpu_sc as plsc

EMBED = 384
NH = 8
NP = 12
NL = 8
HS = 32
WS = 32
HW = HS * WS
DPH = EMBED // NH
HD = DPH // NH
G = 12
IMGS = G * NH
F32 = jnp.float32

NSO = NH * NP * 2
NAWP = NH * 16
NVP = NH * 8

PW = 36
PIMG = PW * PW
PTAB = HD * PIMG


def _proj_body(x_ref, xv_ref, pos_ref, w1_ref, b1_ref, w2_ref, b2_ref,
               so_ref, aw_ref, v_ref):
    q = x_ref[0] + pos_ref[0]
    o1 = jnp.dot(w1_ref[0], q, preferred_element_type=F32) + b1_ref[0]
    so_ref[0] = o1[:NSO]
    a = o1[NSO:]
    a = a - jnp.max(a, axis=0, keepdims=True)
    ex = jnp.exp(a)
    j = lax.broadcasted_iota(jnp.int32, (NH, NAWP), 1)
    i = lax.broadcasted_iota(jnp.int32, (NH, NAWP), 0)
    hsel = ((j // 16 == i) & (j % 16 < NP)).astype(F32)
    rinv = 1.0 / jnp.dot(hsel, ex, preferred_element_type=F32)
    aw_ref[0] = ex * jnp.dot(hsel.T, rinv, preferred_element_type=F32)
    v_ref[0] = jnp.dot(w2_ref[0], xv_ref[0], preferred_element_type=F32) + b2_ref[0]


def _out_body(s_ref, x_ref, w_ref, b_ref, o_ref):
    o_ref[0] = (jnp.dot(w_ref[0], s_ref[0], preferred_element_type=F32)
                + b_ref[0] + x_ref[0])


def _proj_call(x, posT, w1, b1, w2, b2):
    n1 = NSO + NAWP
    return pl.pallas_call(
        _proj_body,
        grid=(G,),
        in_specs=[
            pl.BlockSpec((1, EMBED, HW), lambda g: (g, 0, 0)),
            pl.BlockSpec((1, EMBED, HW), lambda g: ((g + 6) % 12, 0, 0)),
            pl.BlockSpec((1, EMBED, HW), lambda g: (g // 6, 0, 0)),
            pl.BlockSpec((1, n1, EMBED), lambda g: (g // 6, 0, 0)),
            pl.BlockSpec((1, n1, 1), lambda g: (g // 6, 0, 0)),
            pl.BlockSpec((1, NVP, EMBED), lambda g: (g // 6, 0, 0)),
            pl.BlockSpec((1, NVP, 1), lambda g: (g // 6, 0, 0)),
        ],
        out_specs=[
            pl.BlockSpec((1, NSO, HW), lambda g: (g, 0, 0)),
            pl.BlockSpec((1, NAWP, HW), lambda g: (g, 0, 0)),
            pl.BlockSpec((1, NVP, HW), lambda g: (g, 0, 0)),
        ],
        out_shape=[
            jax.ShapeDtypeStruct((G, NSO, HW), F32),
            jax.ShapeDtypeStruct((G, NAWP, HW), F32),
            jax.ShapeDtypeStruct((G, NVP, HW), F32),
        ],
    )(x, x, posT, w1, b1, w2, b2)


def _out_call(samp, x, w, b):
    return pl.pallas_call(
        _out_body,
        grid=(G,),
        in_specs=[
            pl.BlockSpec((1, DPH, HW), lambda g: (g, 0, 0)),
            pl.BlockSpec((1, EMBED, HW), lambda g: (g, 0, 0)),
            pl.BlockSpec((1, EMBED, DPH), lambda g: (g // 6, 0, 0)),
            pl.BlockSpec((1, EMBED, 1), lambda g: (g // 6, 0, 0)),
        ],
        out_specs=pl.BlockSpec((1, EMBED, HW), lambda g: (g, 0, 0)),
        out_shape=jax.ShapeDtypeStruct((G, EMBED, HW), F32),
    )(samp, x, w, b)


@functools.lru_cache(maxsize=None)
def _sc_sampler():
    info = plsc.get_sparse_core_info()
    NC, NS, L = info.num_cores, info.num_subcores, info.num_lanes
    NW = NC * NS
    per_w = IMGS // NW
    nblk = HW // L
    mesh = plsc.VectorSubcoreMesh(core_axis_name="c", subcore_axis_name="s")

    @functools.partial(
        pl.kernel,
        mesh=mesh,
        out_type=jax.ShapeDtypeStruct((IMGS, HD * HW), F32),
        compiler_params=pltpu.CompilerParams(needs_layout_passes=False,
                                             disable_bounds_checks=True),
        scratch_types=[
            pltpu.VMEM((PTAB,), F32),
            pltpu.VMEM((NVP // NH, HW), F32),
            pltpu.VMEM((NP * 2, HW), F32),
            pltpu.VMEM((NP * 2, HW), F32),
            pltpu.VMEM((16, HW), F32),
            pltpu.VMEM((16, HW), F32),
            pltpu.VMEM((HD * HW,), F32),
            pltpu.SemaphoreType.DMA,
            pltpu.SemaphoreType.DMA,
            pltpu.SemaphoreType.DMA,
        ],
    )
    def _sc_sample(so_hbm, aw_hbm, vr_hbm, out_hbm,
                   v_pad, vr_v, so_0, so_1, aw_0, aw_1, o_v,
                   sem0, sem1, semv):
        wid = lax.axis_index("s") * NC + lax.axis_index("c")
        lane = lax.iota(jnp.int32, L)
        bufs = ((so_0, aw_0, sem0), (so_1, aw_1, sem1))

        def zblk(i, c):
            v_pad[pl.ds(i * L, L)] = jnp.zeros((L,), F32)
            return c

        lax.fori_loop(0, PTAB // L, zblk, 0)

        def start(k):
            m = wid * per_w + k
            sb, ab, sem = bufs[k % 2]
            return (pltpu.async_copy(so_hbm.at[m], sb, sem),
                    pltpu.async_copy(aw_hbm.at[m], ab, sem))

        def startv(k):
            return (pltpu.async_copy(vr_hbm.at[wid * per_w + k], vr_v, semv),)

        pend = {0: start(0)}
        pendv = {0: startv(0)}
        for k in range(per_w):
            so_v, aw_v, _ = bufs[k % 2]
            if k + 1 < per_w:
                pend[k + 1] = start(k + 1)
            for hc in pendv.pop(k):
                hc.wait()

            def prow(r, c0, vr_v=vr_v):
                for c in range(HD):
                    for j in range(2):
                        src = vr_v[c, pl.ds(r * WS + j * L, L)]
                        didx = (r * PW
                                + (c * PIMG + PW + 1 + j * L) + lane)
                        plsc.store_scatter(v_pad, [didx], src)
                return c0

            lax.fori_loop(0, HS, prow, 0)
            if k + 1 < per_w:
                pendv[k + 1] = startv(k + 1)
            for hc in pend.pop(k):
                hc.wait()

            taps = [v_pad.at[pl.ds(c * PIMG, PIMG)] for c in range(HD)]

            def blk(i, carry2, so_v=so_v, aw_v=aw_v, taps=taps):
                base = i * L
                rowf = (base // WS).astype(F32)
                colf = ((base % WS) + lane).astype(F32)
                acc = [jnp.zeros((L,), F32) for _ in range(HD)]
                for p in range(NP):
                    ep = aw_v[p, pl.ds(base, L)]
                    xs = jnp.minimum(jnp.maximum(
                        colf + so_v[2 * p, pl.ds(base, L)] + 1.0,
                        0.0), 33.0)
                    ys = jnp.minimum(jnp.maximum(
                        rowf + so_v[2 * p + 1, pl.ds(base, L)] + 1.0,
                        0.0), 33.0)
                    x0i = xs.astype(jnp.int32)
                    y0i = ys.astype(jnp.int32)
                    fx1 = xs - x0i.astype(F32)
                    fx0 = 1.0 - fx1
                    fy1 = ys - y0i.astype(F32)
                    fy0 = 1.0 - fy1
                    pix = y0i * PW + x0i
                    idxs = (pix, pix + 1, pix + PW, pix + (PW + 1))
                    ex0 = fx0 * ep
                    ex1 = fx1 * ep
                    wts = (ex0 * fy0, ex1 * fy0, ex0 * fy1, ex1 * fy1)
                    for c in range(HD):
                        a = acc[c]
                        for t in range(4):
                            a = a + wts[t] * plsc.load_gather(taps[c],
                                                              [idxs[t]])
                        acc[c] = a
                for c in range(HD):
                    o_v[pl.ds(c * HW + base, L)] = acc[c]
                return carry2

            lax.fori_loop(0, nblk, blk, 0)
            pltpu.sync_copy(o_v, out_hbm.at[wid * per_w + k])

    return _sc_sample


def _posT_one(re_s, ce_s):
    pe = jnp.concatenate([
        jnp.broadcast_to(ce_s[None, :, :], (HS, WS, EMBED // 2)),
        jnp.broadcast_to(re_s[:, None, :], (HS, WS, EMBED // 2))], -1)
    return pe.reshape(HW, EMBED).T


def kernel(rgb_fea, ir_fea, so_W, so_b, aw_W, aw_b, vp_W, vp_b, op_W, op_b,
           row_embed, col_embed):
    xr = rgb_fea.transpose(0, 2, 1, 3, 4).reshape(6, EMBED, HW)
    xi = ir_fea.transpose(0, 2, 1, 3, 4).reshape(6, EMBED, HW)
    x = jnp.concatenate([xr, xi], axis=0)

    posT = jnp.stack([_posT_one(row_embed[0], col_embed[0]),
                      _posT_one(row_embed[1], col_embed[1])])

    aw_Wp = jnp.pad(aw_W.reshape(2, NL, EMBED, NH, NP),
                    ((0, 0),) * 4 + ((0, 4),)).reshape(2, NL, EMBED, NAWP)
    aw_bp = jnp.pad(aw_b.reshape(2, NL, NH, NP),
                    ((0, 0),) * 3 + ((0, 4),)).reshape(2, NL, NAWP)
    vp_Wp = jnp.pad(vp_W.reshape(2, NL, EMBED, NH, HD),
                    ((0, 0),) * 4 + ((0, 2),)).reshape(2, NL, EMBED, NVP)
    vp_bp = jnp.pad(vp_b.reshape(2, NL, NH, HD),
                    ((0, 0),) * 3 + ((0, 2),)).reshape(2, NL, NVP)
    w1 = jnp.swapaxes(jnp.concatenate([so_W, aw_Wp], -1), -1, -2)
    b1 = jnp.concatenate([so_b, aw_bp], -1)[..., None]
    w2 = jnp.swapaxes(vp_Wp, -1, -2)
    b2 = vp_bp[..., None]
    w3 = jnp.swapaxes(op_W, -1, -2)
    b3 = op_b[..., None]

    sampler = _sc_sampler()
    for l in range(NL):
        so_t, aw_t, v_t = _proj_call(x, posT, w1[:, l], b1[:, l],
                                     w2[:, l], b2[:, l])
        samp = sampler(so_t.reshape(IMGS, NP * 2, HW),
                       aw_t.reshape(IMGS, 16, HW),
                       v_t.reshape(IMGS, NVP // NH, HW))
        x = _out_call(samp.reshape(G, DPH, HW), x, w3[:, l], b3[:, l])

    y = x.reshape(2, 2, 3, EMBED, HS, WS).transpose(0, 1, 3, 2, 4, 5)
    return y


# speedup vs baseline: 1.5581x; 1.0829x over previous
import functools

import jax
import jax.numpy as jnp
from jax import lax
from jax.experimental import pallas as pl
from jax.experimental.pallas import tpu as pltpu
from jax.experimental.pallas import tpu_sc as plsc

EMBED = 384
NH = 8
NP = 12
NL = 8
HS = 32
WS = 32
HW = HS * WS
DPH = EMBED // NH
HD = DPH // NH
G = 12
IMGS = G * NH
F32 = jnp.float32

NSO = NH * NP * 2
NAWP = NH * 16
NVP = NH * 8

PW = 36
PIMG = PW * PW
PTAB = HD * PIMG


def _proj_body(x_ref, xv_ref, pos_ref, w1_ref, b1_ref, w2_ref, b2_ref,
               so_ref, aw_ref, v_ref):
    q = x_ref[0] + pos_ref[0]
    o1 = jnp.dot(w1_ref[0], q, preferred_element_type=F32) + b1_ref[0]
    so_ref[0] = o1[:NSO]
    a = o1[NSO:]
    a = a - jnp.max(a, axis=0, keepdims=True)
    ex = jnp.exp(a)
    j = lax.broadcasted_iota(jnp.int32, (NH, NAWP), 1)
    i = lax.broadcasted_iota(jnp.int32, (NH, NAWP), 0)
    hsel = ((j // 16 == i) & (j % 16 < NP)).astype(F32)
    rinv = 1.0 / jnp.dot(hsel, ex, preferred_element_type=F32)
    aw_ref[0] = ex * jnp.dot(hsel.T, rinv, preferred_element_type=F32)
    v_ref[0] = jnp.dot(w2_ref[0], xv_ref[0], preferred_element_type=F32) + b2_ref[0]


def _out_body(s_ref, x_ref, w_ref, b_ref, o_ref):
    o_ref[0] = (jnp.dot(w_ref[0], s_ref[0], preferred_element_type=F32)
                + b_ref[0] + x_ref[0])


def _fused_body(sg_ref, so_ref_in, xg_ref, xo_ref, w3g_ref, b3g_ref,
                w3o_ref, b3o_ref, pos_ref, w1_ref, b1_ref, w2_ref, b2_ref,
                xn_ref, so_ref, aw_ref, v_ref):
    xn = (jnp.dot(w3g_ref[0], sg_ref[0], preferred_element_type=F32)
          + b3g_ref[0] + xg_ref[0])
    xn_ref[0] = xn
    xo = (jnp.dot(w3o_ref[0], so_ref_in[0], preferred_element_type=F32)
          + b3o_ref[0] + xo_ref[0])
    q = xn + pos_ref[0]
    o1 = jnp.dot(w1_ref[0], q, preferred_element_type=F32) + b1_ref[0]
    so_ref[0] = o1[:NSO]
    a = o1[NSO:]
    a = a - jnp.max(a, axis=0, keepdims=True)
    ex = jnp.exp(a)
    j = lax.broadcasted_iota(jnp.int32, (NH, NAWP), 1)
    i = lax.broadcasted_iota(jnp.int32, (NH, NAWP), 0)
    hsel = ((j // 16 == i) & (j % 16 < NP)).astype(F32)
    rinv = 1.0 / jnp.dot(hsel, ex, preferred_element_type=F32)
    aw_ref[0] = ex * jnp.dot(hsel.T, rinv, preferred_element_type=F32)
    v_ref[0] = jnp.dot(w2_ref[0], xo, preferred_element_type=F32) + b2_ref[0]


def _fused_call(samp, x, w3, b3, posT, w1, b1, w2, b2):
    n1 = NSO + NAWP
    return pl.pallas_call(
        _fused_body,
        grid=(G,),
        in_specs=[
            pl.BlockSpec((1, DPH, HW), lambda g: (g, 0, 0)),
            pl.BlockSpec((1, DPH, HW), lambda g: ((g + 6) % 12, 0, 0)),
            pl.BlockSpec((1, EMBED, HW), lambda g: (g, 0, 0)),
            pl.BlockSpec((1, EMBED, HW), lambda g: ((g + 6) % 12, 0, 0)),
            pl.BlockSpec((1, EMBED, DPH), lambda g: (g // 6, 0, 0)),
            pl.BlockSpec((1, EMBED, 1), lambda g: (g // 6, 0, 0)),
            pl.BlockSpec((1, EMBED, DPH), lambda g: ((g // 6 + 1) % 2, 0, 0)),
            pl.BlockSpec((1, EMBED, 1), lambda g: ((g // 6 + 1) % 2, 0, 0)),
            pl.BlockSpec((1, EMBED, HW), lambda g: (g // 6, 0, 0)),
            pl.BlockSpec((1, n1, EMBED), lambda g: (g // 6, 0, 0)),
            pl.BlockSpec((1, n1, 1), lambda g: (g // 6, 0, 0)),
            pl.BlockSpec((1, NVP, EMBED), lambda g: (g // 6, 0, 0)),
            pl.BlockSpec((1, NVP, 1), lambda g: (g // 6, 0, 0)),
        ],
        out_specs=[
            pl.BlockSpec((1, EMBED, HW), lambda g: (g, 0, 0)),
            pl.BlockSpec((1, NSO, HW), lambda g: (g, 0, 0)),
            pl.BlockSpec((1, NAWP, HW), lambda g: (g, 0, 0)),
            pl.BlockSpec((1, NVP, HW), lambda g: (g, 0, 0)),
        ],
        out_shape=[
            jax.ShapeDtypeStruct((G, EMBED, HW), F32),
            jax.ShapeDtypeStruct((G, NSO, HW), F32),
            jax.ShapeDtypeStruct((G, NAWP, HW), F32),
            jax.ShapeDtypeStruct((G, NVP, HW), F32),
        ],
    )(samp, samp, x, x, w3, b3, w3, b3, posT, w1, b1, w2, b2)


def _proj_call(x, posT, w1, b1, w2, b2):
    n1 = NSO + NAWP
    return pl.pallas_call(
        _proj_body,
        grid=(G,),
        in_specs=[
            pl.BlockSpec((1, EMBED, HW), lambda g: (g, 0, 0)),
            pl.BlockSpec((1, EMBED, HW), lambda g: ((g + 6) % 12, 0, 0)),
            pl.BlockSpec((1, EMBED, HW), lambda g: (g // 6, 0, 0)),
            pl.BlockSpec((1, n1, EMBED), lambda g: (g // 6, 0, 0)),
            pl.BlockSpec((1, n1, 1), lambda g: (g // 6, 0, 0)),
            pl.BlockSpec((1, NVP, EMBED), lambda g: (g // 6, 0, 0)),
            pl.BlockSpec((1, NVP, 1), lambda g: (g // 6, 0, 0)),
        ],
        out_specs=[
            pl.BlockSpec((1, NSO, HW), lambda g: (g, 0, 0)),
            pl.BlockSpec((1, NAWP, HW), lambda g: (g, 0, 0)),
            pl.BlockSpec((1, NVP, HW), lambda g: (g, 0, 0)),
        ],
        out_shape=[
            jax.ShapeDtypeStruct((G, NSO, HW), F32),
            jax.ShapeDtypeStruct((G, NAWP, HW), F32),
            jax.ShapeDtypeStruct((G, NVP, HW), F32),
        ],
    )(x, x, posT, w1, b1, w2, b2)


def _out_call(samp, x, w, b):
    return pl.pallas_call(
        _out_body,
        grid=(G,),
        in_specs=[
            pl.BlockSpec((1, DPH, HW), lambda g: (g, 0, 0)),
            pl.BlockSpec((1, EMBED, HW), lambda g: (g, 0, 0)),
            pl.BlockSpec((1, EMBED, DPH), lambda g: (g // 6, 0, 0)),
            pl.BlockSpec((1, EMBED, 1), lambda g: (g // 6, 0, 0)),
        ],
        out_specs=pl.BlockSpec((1, EMBED, HW), lambda g: (g, 0, 0)),
        out_shape=jax.ShapeDtypeStruct((G, EMBED, HW), F32),
    )(samp, x, w, b)


@functools.lru_cache(maxsize=None)
def _sc_sampler():
    info = plsc.get_sparse_core_info()
    NC, NS, L = info.num_cores, info.num_subcores, info.num_lanes
    NW = NC * NS
    per_w = IMGS // NW
    nblk = HW // L
    mesh = plsc.VectorSubcoreMesh(core_axis_name="c", subcore_axis_name="s")

    @functools.partial(
        pl.kernel,
        mesh=mesh,
        out_type=jax.ShapeDtypeStruct((IMGS, HD * HW), F32),
        compiler_params=pltpu.CompilerParams(needs_layout_passes=False,
                                             disable_bounds_checks=True),
        scratch_types=[
            pltpu.VMEM((PTAB,), F32),
            pltpu.VMEM((NVP // NH, HW), F32),
            pltpu.VMEM((NP * 2, HW), F32),
            pltpu.VMEM((NP * 2, HW), F32),
            pltpu.VMEM((16, HW), F32),
            pltpu.VMEM((16, HW), F32),
            pltpu.VMEM((HD * HW,), F32),
            pltpu.SemaphoreType.DMA,
            pltpu.SemaphoreType.DMA,
            pltpu.SemaphoreType.DMA,
        ],
    )
    def _sc_sample(so_hbm, aw_hbm, vr_hbm, out_hbm,
                   v_pad, vr_v, so_0, so_1, aw_0, aw_1, o_v,
                   sem0, sem1, semv):
        wid = lax.axis_index("s") * NC + lax.axis_index("c")
        lane = lax.iota(jnp.int32, L)
        bufs = ((so_0, aw_0, sem0), (so_1, aw_1, sem1))

        def zblk(i, c):
            v_pad[pl.ds(i * L, L)] = jnp.zeros((L,), F32)
            return c

        lax.fori_loop(0, PTAB // L, zblk, 0)

        def start(k):
            m = wid * per_w + k
            sb, ab, sem = bufs[k % 2]
            return (pltpu.async_copy(so_hbm.at[m], sb, sem),
                    pltpu.async_copy(aw_hbm.at[m], ab, sem))

        def startv(k):
            return (pltpu.async_copy(vr_hbm.at[wid * per_w + k], vr_v, semv),)

        pend = {0: start(0)}
        pendv = {0: startv(0)}
        for k in range(per_w):
            so_v, aw_v, _ = bufs[k % 2]
            if k + 1 < per_w:
                pend[k + 1] = start(k + 1)
            for hc in pendv.pop(k):
                hc.wait()

            def prow(r, c0, vr_v=vr_v):
                for c in range(HD):
                    for j in range(2):
                        src = vr_v[c, pl.ds(r * WS + j * L, L)]
                        didx = (r * PW
                                + (c * PIMG + PW + 1 + j * L) + lane)
                        plsc.store_scatter(v_pad, [didx], src)
                return c0

            lax.fori_loop(0, HS, prow, 0)
            if k + 1 < per_w:
                pendv[k + 1] = startv(k + 1)
            for hc in pend.pop(k):
                hc.wait()

            taps = [v_pad.at[pl.ds(c * PIMG, PIMG)] for c in range(HD)]

            def blk(i, carry2, so_v=so_v, aw_v=aw_v, taps=taps):
                base = i * L
                rowf = (base // WS).astype(F32)
                colf = ((base % WS) + lane).astype(F32)
                acc = [jnp.zeros((L,), F32) for _ in range(HD)]
                for p in range(NP):
                    ep = aw_v[p, pl.ds(base, L)]
                    xs = jnp.minimum(jnp.maximum(
                        colf + so_v[2 * p, pl.ds(base, L)] + 1.0,
                        0.0), 33.0)
                    ys = jnp.minimum(jnp.maximum(
                        rowf + so_v[2 * p + 1, pl.ds(base, L)] + 1.0,
                        0.0), 33.0)
                    x0i = xs.astype(jnp.int32)
                    y0i = ys.astype(jnp.int32)
                    fx1 = xs - x0i.astype(F32)
                    fx0 = 1.0 - fx1
                    fy1 = ys - y0i.astype(F32)
                    fy0 = 1.0 - fy1
                    pix = y0i * PW + x0i
                    idxs = (pix, pix + 1, pix + PW, pix + (PW + 1))
                    ex0 = fx0 * ep
                    ex1 = fx1 * ep
                    wts = (ex0 * fy0, ex1 * fy0, ex0 * fy1, ex1 * fy1)
                    for c in range(HD):
                        a = acc[c]
                        for t in range(4):
                            a = a + wts[t] * plsc.load_gather(taps[c],
                                                              [idxs[t]])
                        acc[c] = a
                for c in range(HD):
                    o_v[pl.ds(c * HW + base, L)] = acc[c]
                return carry2

            lax.fori_loop(0, nblk, blk, 0)
            pltpu.sync_copy(o_v, out_hbm.at[wid * per_w + k])

    return _sc_sample


def _posT_one(re_s, ce_s):
    pe = jnp.concatenate([
        jnp.broadcast_to(ce_s[None, :, :], (HS, WS, EMBED // 2)),
        jnp.broadcast_to(re_s[:, None, :], (HS, WS, EMBED // 2))], -1)
    return pe.reshape(HW, EMBED).T


def kernel(rgb_fea, ir_fea, so_W, so_b, aw_W, aw_b, vp_W, vp_b, op_W, op_b,
           row_embed, col_embed):
    xr = rgb_fea.transpose(0, 2, 1, 3, 4).reshape(6, EMBED, HW)
    xi = ir_fea.transpose(0, 2, 1, 3, 4).reshape(6, EMBED, HW)
    x = jnp.concatenate([xr, xi], axis=0)

    posT = jnp.stack([_posT_one(row_embed[0], col_embed[0]),
                      _posT_one(row_embed[1], col_embed[1])])

    aw_Wp = jnp.pad(aw_W.reshape(2, NL, EMBED, NH, NP),
                    ((0, 0),) * 4 + ((0, 4),)).reshape(2, NL, EMBED, NAWP)
    aw_bp = jnp.pad(aw_b.reshape(2, NL, NH, NP),
                    ((0, 0),) * 3 + ((0, 4),)).reshape(2, NL, NAWP)
    vp_Wp = jnp.pad(vp_W.reshape(2, NL, EMBED, NH, HD),
                    ((0, 0),) * 4 + ((0, 2),)).reshape(2, NL, EMBED, NVP)
    vp_bp = jnp.pad(vp_b.reshape(2, NL, NH, HD),
                    ((0, 0),) * 3 + ((0, 2),)).reshape(2, NL, NVP)
    w1 = jnp.swapaxes(jnp.concatenate([so_W, aw_Wp], -1), -1, -2)
    b1 = jnp.concatenate([so_b, aw_bp], -1)[..., None]
    w2 = jnp.swapaxes(vp_Wp, -1, -2)
    b2 = vp_bp[..., None]
    w3 = jnp.swapaxes(op_W, -1, -2)
    b3 = op_b[..., None]

    sampler = _sc_sampler()

    def run_sc(so_t, aw_t, v_t):
        return sampler(so_t.reshape(IMGS, NP * 2, HW),
                       aw_t.reshape(IMGS, 16, HW),
                       v_t.reshape(IMGS, NVP // NH, HW)).reshape(G, DPH, HW)

    so_t, aw_t, v_t = _proj_call(x, posT, w1[:, 0], b1[:, 0],
                                 w2[:, 0], b2[:, 0])
    samp = run_sc(so_t, aw_t, v_t)
    for l in range(1, NL):
        x, so_t, aw_t, v_t = _fused_call(samp, x, w3[:, l - 1], b3[:, l - 1],
                                         posT, w1[:, l], b1[:, l],
                                         w2[:, l], b2[:, l])
        samp = run_sc(so_t, aw_t, v_t)
    x = _out_call(samp, x, w3[:, NL - 1], b3[:, NL - 1])

    y = x.reshape(2, 2, 3, EMBED, HS, WS).transpose(0, 1, 3, 2, 4, 5)
    return y
